# baseline (speedup 1.0000x reference)
import functools

import jax
import jax.numpy as jnp
from jax import lax
from jax.experimental import pallas as pl
from jax.experimental.pallas import tpu as pltpu
from jax.experimental.pallas import tpu_sc as plsc

N = 10000
D = 128
E = 320000

NC = 2
NS = 16
NW = NC * NS
EPT = E // NW
C = 80
NCH = EPT // C
CS = 128
NCHS = 79
EPAD = NW * NCHS * CS - E
NPAD = 10240
RPT = NPAD // NS
ZCP = RPT // C

_f32 = jnp.float32


def _sc_mesh():
    return plsc.VectorSubcoreMesh(core_axis_name="c", subcore_axis_name="s",
                                  num_cores=NC, num_subcores=NS)


def _scat_body(table_hbm, eidx_hbm, zeros_hbm,
               out_hbm,
               idx_v, rows_v, isem, gsem,
               acc):
    c = lax.axis_index("c")
    s = lax.axis_index("s")
    wid = s * NC + c
    base = s * RPT

    for j in range(RPT // CS):
        pltpu.sync_copy(zeros_hbm, acc.at[pl.ds(base + j * CS, CS)])
    plsc.subcore_barrier()

    pltpu.async_copy(eidx_hbm.at[wid, 0], idx_v.at[0], isem)

    def chunk(j, carry):
        slot = lax.rem(j, 2)
        pslot = lax.rem(j + 1, 2)
        pltpu.make_async_copy(eidx_hbm.at[wid, j], idx_v.at[slot], isem).wait()
        pltpu.async_copy(table_hbm.at[idx_v.at[slot, 0]], rows_v.at[slot], gsem)

        @pl.when(j > 0)
        def _():
            pltpu.make_async_copy(table_hbm.at[idx_v.at[pslot, 0]],
                                  rows_v.at[pslot], gsem).wait()
            pltpu.sync_copy(rows_v.at[pslot], acc.at[idx_v.at[pslot, 1]],
                            add=True)

        @pl.when(j + 1 < NCHS)
        def _():
            pltpu.async_copy(eidx_hbm.at[wid, j + 1], idx_v.at[pslot], isem)

        return carry

    lax.fori_loop(0, NCHS, chunk, 0)
    lslot = (NCHS - 1) % 2
    pltpu.make_async_copy(table_hbm.at[idx_v.at[lslot, 0]],
                          rows_v.at[lslot], gsem).wait()
    pltpu.sync_copy(rows_v.at[lslot], acc.at[idx_v.at[lslot, 1]], add=True)

    plsc.subcore_barrier()
    pltpu.sync_copy(acc.at[pl.ds(base, RPT)],
                    out_hbm.at[pl.ds(c * NPAD + base, RPT)])


def _scat_call(table, eidx, zeros128):
    f = pl.kernel(
        _scat_body,
        out_type=jax.ShapeDtypeStruct((NC * NPAD, D), _f32),
        mesh=_sc_mesh(),
        scratch_types=[
            pltpu.VMEM((2, 2, CS), jnp.int32),
            pltpu.VMEM((2, CS, D), _f32),
            pltpu.SemaphoreType.DMA,
            pltpu.SemaphoreType.DMA,
            pltpu.VMEM_SHARED((NPAD, D), _f32),
        ],
    )
    return f(table, eidx, zeros128)


HB = NPAD // 2
HBP = HB + 16
NV = EPT // 16
NRED = HB // 16


def _deg_body(ei_hbm, out_hbm, sidx_v, didx_v, hist_v, red_v):
    c = lax.axis_index("c")
    s = lax.axis_index("s")
    wid = s * NC + c

    pltpu.sync_copy(ei_hbm.at[pl.ds(wid * EPT, EPT)], sidx_v)
    pltpu.sync_copy(ei_hbm.at[pl.ds(E + wid * EPT, EPT)], didx_v)

    lane = lax.iota(jnp.int32, 16)
    ones16 = jnp.ones((16,), _f32)
    zeros16 = jnp.zeros((16,), _f32)

    for d in range(2):
        idx_ref = sidx_v if d == 0 else didx_v
        for h in range(2):
            lo = h * HB

            def zero(i, carry):
                for u in range(8):
                    hist_v[pl.ds(i * 128 + u * 16, 16)] = zeros16
                return carry

            lax.fori_loop(0, HBP * 16 // 128, zero, 0)

            def cnt(i, carry):
                idx = idx_ref[pl.ds(i * 16, 16)]
                m = (idx >= lo) & (idx < lo + HB)
                pos = lane * HBP + jnp.where(m, idx - lo, HB)
                cur = plsc.load_gather(hist_v, [pos])
                plsc.store_scatter(hist_v, [pos], cur + ones16)
                return carry

            lax.fori_loop(0, NV, cnt, 0)

            def red(i, carry):
                vals = [hist_v[pl.ds(cc * HBP + i * 16, 16)]
                        for cc in range(16)]
                while len(vals) > 1:
                    vals = [vals[k] + vals[k + 1]
                            for k in range(0, len(vals), 2)]
                red_v[pl.ds(i * 16, 16)] = vals[0]
                return carry

            lax.fori_loop(0, NRED, red, 0)
            pltpu.sync_copy(red_v, out_hbm.at[pl.ds((d * NW + wid) * NPAD + lo, HB)])


def _deg_call(edge_index):
    f = pl.kernel(
        _deg_body,
        out_type=jax.ShapeDtypeStruct((2 * NW * NPAD,), _f32),
        mesh=_sc_mesh(),
        compiler_params=pltpu.CompilerParams(needs_layout_passes=False),
        scratch_types=[
            pltpu.VMEM((EPT,), jnp.int32),
            pltpu.VMEM((EPT,), jnp.int32),
            pltpu.VMEM((HBP * 16,), _f32),
            pltpu.VMEM((HB,), _f32),
        ],
    )
    return f(edge_index.reshape(2 * E)).reshape(2, NW, NPAD)


def _prep_body(x_ref, dego_ref, out_ref):
    dego = jnp.sum(dego_ref[...], axis=0) + 1.0
    nsrc = lax.rsqrt(dego)
    out_ref[...] = x_ref[...] * nsrc[:, None]


def _prep_call(x, dego_p):
    return pl.pallas_call(
        _prep_body,
        out_shape=jax.ShapeDtypeStruct((N, D), _f32),
    )(x, dego_p)


def _dense_mid_body(sp_ref, hp_ref, dego_ref, degi_ref, w_ref, b_ref,
                    g_ref, bb_ref, out_ref):
    degi = jnp.sum(degi_ref[...], axis=0) + 1.0
    ndst = lax.rsqrt(degi)
    agg = (sp_ref[0, :N, :] + sp_ref[1, :N, :] + hp_ref[...]) * ndst[:, None]
    z = jnp.dot(agg, w_ref[...], preferred_element_type=_f32) + b_ref[...][None, :]
    r = jnp.maximum(z, 0.0)
    m = jnp.mean(r, axis=0)
    v = jnp.mean(r * r, axis=0) - m * m
    h = (r - m) * lax.rsqrt(v + 1e-5) * g_ref[...][None, :] + bb_ref[...][None, :]
    dego = jnp.sum(dego_ref[...], axis=0) + 1.0
    nsrc = lax.rsqrt(dego)
    out_ref[...] = h * nsrc[:, None]


def _dense_mid_call(sp, hp, dego_p, degi_p, w, b, g, bb):
    sp = sp.reshape(NC, NPAD, D)
    return pl.pallas_call(
        _dense_mid_body,
        out_shape=jax.ShapeDtypeStruct((N, D), _f32),
    )(sp, hp, dego_p, degi_p, w, b, g, bb)


def _dense_fin_body(sp_ref, hp_ref, degi_ref, w_ref, b_ref,
                    g_ref, bb_ref, out_ref):
    degi = jnp.sum(degi_ref[...], axis=0) + 1.0
    ndst = lax.rsqrt(degi)
    agg = (sp_ref[0, :N, :] + sp_ref[1, :N, :] + hp_ref[...]) * ndst[:, None]
    z = jnp.dot(agg, w_ref[...], preferred_element_type=_f32) + b_ref[...][None, :]
    m = jnp.mean(z, axis=-1, keepdims=True)
    zc = z - m
    v = jnp.mean(zc * zc, axis=-1, keepdims=True)
    out_ref[...] = zc * lax.rsqrt(v + 1e-5) * g_ref[...][None, :] + bb_ref[...][None, :]


def _dense_fin_call(sp, hp, degi_p, w, b, g, bb):
    sp = sp.reshape(NC, NPAD, D)
    return pl.pallas_call(
        _dense_fin_body,
        out_shape=jax.ShapeDtypeStruct((N, D), _f32),
    )(sp, hp, degi_p, w, b, g, bb)


def kernel(x, edge_index, W1, b1, W2, b2, W3, b3,
           bn1_g, bn1_b, bn2_g, bn2_b, ln_g, ln_b):
    src_p = jnp.concatenate([edge_index[0], jnp.zeros((EPAD,), jnp.int32)])
    dst_p = jnp.concatenate([edge_index[1],
                             jnp.full((EPAD,), N, jnp.int32)])
    eidx = jnp.stack([src_p.reshape(NW, NCHS, CS),
                      dst_p.reshape(NW, NCHS, CS)], axis=2)
    zeros128 = jnp.zeros((CS, D), _f32)

    deg_p = _deg_call(edge_index)
    dego_p = deg_p[0, :, :N]
    degi_p = deg_p[1, :, :N]

    h1p = _prep_call(x, dego_p)
    s1 = _scat_call(h1p, eidx, zeros128)
    h2p = _dense_mid_call(s1, h1p, dego_p, degi_p, W1, b1, bn1_g, bn1_b)
    s2 = _scat_call(h2p, eidx, zeros128)
    h3p = _dense_mid_call(s2, h2p, dego_p, degi_p, W2, b2, bn2_g, bn2_b)
    s3 = _scat_call(h3p, eidx, zeros128)
    return _dense_fin_call(s3, h3p, degi_p, W3, b3, ln_g, ln_b)


# speedup vs baseline: 1.0005x; 1.0005x over previous
import functools

import jax
import jax.numpy as jnp
from jax import lax
from jax.experimental import pallas as pl
from jax.experimental.pallas import tpu as pltpu
from jax.experimental.pallas import tpu_sc as plsc

N = 10000
D = 128
E = 320000

NC = 2
NS = 16
NW = NC * NS
EPT = E // NW
C = 80
NCH = EPT // C
CS = 128
NCHS = 79
EPAD = NW * NCHS * CS - E
NPAD = 10240
RPT = NPAD // NS
ZCP = RPT // C

_f32 = jnp.float32


def _sc_mesh():
    return plsc.VectorSubcoreMesh(core_axis_name="c", subcore_axis_name="s",
                                  num_cores=NC, num_subcores=NS)


def _scat_body(table_hbm, eidx_hbm, zeros_hbm,
               out_hbm,
               idx_v, rows_v, isem, gsem,
               acc):
    c = lax.axis_index("c")
    s = lax.axis_index("s")
    wid = s * NC + c
    base = s * RPT

    for j in range(RPT // CS):
        pltpu.sync_copy(zeros_hbm, acc.at[pl.ds(base + j * CS, CS)])
    plsc.subcore_barrier()

    pltpu.async_copy(eidx_hbm.at[wid, 0], idx_v.at[0], isem)

    def chunk(j, carry):
        slot = lax.rem(j, 2)
        pslot = lax.rem(j + 1, 2)
        pltpu.make_async_copy(eidx_hbm.at[wid, j], idx_v.at[slot], isem).wait()
        pltpu.async_copy(table_hbm.at[idx_v.at[slot, 0]], rows_v.at[slot], gsem)

        @pl.when(j > 0)
        def _():
            pltpu.make_async_copy(table_hbm.at[idx_v.at[pslot, 0]],
                                  rows_v.at[pslot], gsem).wait()
            pltpu.sync_copy(rows_v.at[pslot], acc.at[idx_v.at[pslot, 1]],
                            add=True)

        @pl.when(j + 1 < NCHS)
        def _():
            pltpu.async_copy(eidx_hbm.at[wid, j + 1], idx_v.at[pslot], isem)

        return carry

    lax.fori_loop(0, NCHS, chunk, 0)
    lslot = (NCHS - 1) % 2
    pltpu.make_async_copy(table_hbm.at[idx_v.at[lslot, 0]],
                          rows_v.at[lslot], gsem).wait()
    pltpu.sync_copy(rows_v.at[lslot], acc.at[idx_v.at[lslot, 1]], add=True)

    plsc.subcore_barrier()
    pltpu.sync_copy(acc.at[pl.ds(base, RPT)],
                    out_hbm.at[pl.ds(c * NPAD + base, RPT)])


def _scat_call(table, eidx, zeros128):
    f = pl.kernel(
        _scat_body,
        out_type=jax.ShapeDtypeStruct((NC * NPAD, D), _f32),
        mesh=_sc_mesh(),
        scratch_types=[
            pltpu.VMEM((2, 2, CS), jnp.int32),
            pltpu.VMEM((2, CS, D), _f32),
            pltpu.SemaphoreType.DMA,
            pltpu.SemaphoreType.DMA,
            pltpu.VMEM_SHARED((NPAD, D), _f32),
        ],
    )
    return f(table, eidx, zeros128)


HB = NPAD // 2
HBP = HB + 16
NV = EPT // 16
NRED = HB // 16


def _deg_body(ei_hbm, out_hbm, sidx_v, didx_v, hist_v, red_v):
    c = lax.axis_index("c")
    s = lax.axis_index("s")
    wid = s * NC + c

    pltpu.sync_copy(ei_hbm.at[pl.ds(wid * EPT, EPT)], sidx_v)
    pltpu.sync_copy(ei_hbm.at[pl.ds(E + wid * EPT, EPT)], didx_v)

    lane = lax.iota(jnp.int32, 16)
    ones16 = jnp.ones((16,), _f32)
    zeros16 = jnp.zeros((16,), _f32)

    for d in range(2):
        idx_ref = sidx_v if d == 0 else didx_v
        for h in range(2):
            lo = h * HB

            def zero(i, carry):
                for u in range(8):
                    hist_v[pl.ds(i * 128 + u * 16, 16)] = zeros16
                return carry

            lax.fori_loop(0, HBP * 16 // 128, zero, 0)

            def cnt(i, carry):
                idx = idx_ref[pl.ds(i * 16, 16)]
                m = (idx >= lo) & (idx < lo + HB)
                pos = lane * HBP + jnp.where(m, idx - lo, HB)
                cur = plsc.load_gather(hist_v, [pos])
                plsc.store_scatter(hist_v, [pos], cur + ones16)
                return carry

            lax.fori_loop(0, NV, cnt, 0)

            def red(i, carry):
                vals = [hist_v[pl.ds(cc * HBP + i * 16, 16)]
                        for cc in range(16)]
                while len(vals) > 1:
                    vals = [vals[k] + vals[k + 1]
                            for k in range(0, len(vals), 2)]
                red_v[pl.ds(i * 16, 16)] = vals[0]
                return carry

            lax.fori_loop(0, NRED, red, 0)
            pltpu.sync_copy(red_v, out_hbm.at[pl.ds((d * NW + wid) * NPAD + lo, HB)])


def _deg_call(edge_index):
    f = pl.kernel(
        _deg_body,
        out_type=jax.ShapeDtypeStruct((2 * NW * NPAD,), _f32),
        mesh=_sc_mesh(),
        compiler_params=pltpu.CompilerParams(needs_layout_passes=False),
        scratch_types=[
            pltpu.VMEM((EPT,), jnp.int32),
            pltpu.VMEM((EPT,), jnp.int32),
            pltpu.VMEM((HBP * 16,), _f32),
            pltpu.VMEM((HB,), _f32),
        ],
    )
    return f(edge_index.reshape(2 * E)).reshape(2, NW, NPAD)


def _prep_body(x_ref, dego_ref, out_ref):
    dego = jnp.sum(dego_ref[...], axis=0) + 1.0
    nsrc = lax.rsqrt(dego)
    out_ref[...] = x_ref[...] * nsrc[:, None]


def _prep_call(x, dego_p):
    return pl.pallas_call(
        _prep_body,
        out_shape=jax.ShapeDtypeStruct((N, D), _f32),
    )(x, dego_p)


def _dense_mid_body(sp_ref, hp_ref, dego_ref, degi_ref, w_ref, b_ref,
                    g_ref, bb_ref, out_ref):
    degi = jnp.sum(degi_ref[...], axis=0) + 1.0
    ndst = lax.rsqrt(degi)
    agg = (sp_ref[0, :N, :] + sp_ref[1, :N, :] + hp_ref[...]) * ndst[:, None]
    z = jnp.dot(agg, w_ref[...], preferred_element_type=_f32) + b_ref[...][None, :]
    r = jnp.maximum(z, 0.0)
    m = jnp.mean(r, axis=0)
    v = jnp.mean(r * r, axis=0) - m * m
    h = (r - m) * lax.rsqrt(v + 1e-5) * g_ref[...][None, :] + bb_ref[...][None, :]
    dego = jnp.sum(dego_ref[...], axis=0) + 1.0
    nsrc = lax.rsqrt(dego)
    out_ref[...] = h * nsrc[:, None]


def _dense_mid_call(sp, hp, dego_p, degi_p, w, b, g, bb):
    sp = sp.reshape(NC, NPAD, D)
    return pl.pallas_call(
        _dense_mid_body,
        out_shape=jax.ShapeDtypeStruct((N, D), _f32),
    )(sp, hp, dego_p, degi_p, w, b, g, bb)


def _dense_fin_body(sp_ref, hp_ref, degi_ref, w_ref, b_ref,
                    g_ref, bb_ref, out_ref):
    degi = jnp.sum(degi_ref[...], axis=0) + 1.0
    ndst = lax.rsqrt(degi)
    agg = (sp_ref[0, :N, :] + sp_ref[1, :N, :] + hp_ref[...]) * ndst[:, None]
    z = jnp.dot(agg, w_ref[...], preferred_element_type=_f32) + b_ref[...][None, :]
    m = jnp.mean(z, axis=-1, keepdims=True)
    zc = z - m
    v = jnp.mean(zc * zc, axis=-1, keepdims=True)
    out_ref[...] = zc * lax.rsqrt(v + 1e-5) * g_ref[...][None, :] + bb_ref[...][None, :]


def _dense_fin_call(sp, hp, degi_p, w, b, g, bb):
    sp = sp.reshape(NC, NPAD, D)
    return pl.pallas_call(
        _dense_fin_body,
        out_shape=jax.ShapeDtypeStruct((N, D), _f32),
    )(sp, hp, degi_p, w, b, g, bb)


def kernel(x, edge_index, W1, b1, W2, b2, W3, b3,
           bn1_g, bn1_b, bn2_g, bn2_b, ln_g, ln_b):
    src_p = jnp.concatenate([edge_index[0], jnp.zeros((EPAD,), jnp.int32)])
    dst_p = jnp.concatenate([edge_index[1],
                             N + jnp.arange(EPAD, dtype=jnp.int32)
                             % (NPAD - N)])
    eidx = jnp.stack([src_p.reshape(NW, NCHS, CS),
                      dst_p.reshape(NW, NCHS, CS)], axis=2)
    zeros128 = jnp.zeros((CS, D), _f32)

    deg_p = _deg_call(edge_index)
    dego_p = deg_p[0, :, :N]
    degi_p = deg_p[1, :, :N]

    h1p = _prep_call(x, dego_p)
    s1 = _scat_call(h1p, eidx, zeros128)
    h2p = _dense_mid_call(s1, h1p, dego_p, degi_p, W1, b1, bn1_g, bn1_b)
    s2 = _scat_call(h2p, eidx, zeros128)
    h3p = _dense_mid_call(s2, h2p, dego_p, degi_p, W2, b2, bn2_g, bn2_b)
    s3 = _scat_call(h3p, eidx, zeros128)
    return _dense_fin_call(s3, h3p, degi_p, W3, b3, ln_g, ln_b)


# speedup vs baseline: 1.9264x; 1.9254x over previous
import functools

import jax
import jax.numpy as jnp
from jax import lax
from jax.experimental import pallas as pl
from jax.experimental.pallas import tpu as pltpu
from jax.experimental.pallas import tpu_sc as plsc

N = 10000
D = 128
E = 320000

NC = 2
NS = 16
NW = NC * NS
EPT = E // NW
C = 80
NCH = EPT // C
CS = 128
NCHS = 79
EPAD = NW * NCHS * CS - E
NPAD = 10240
RPT = NPAD // NS
ZCP = RPT // C

_f32 = jnp.float32


def _sc_mesh():
    return plsc.VectorSubcoreMesh(core_axis_name="c", subcore_axis_name="s",
                                  num_cores=NC, num_subcores=NS)


def _scat_body(table_hbm, eidx_hbm, zeros_hbm,
               out_hbm,
               idx_v, rows_v, isem, gsem,
               acc):
    c = lax.axis_index("c")
    s = lax.axis_index("s")
    wid = s * NC + c
    base = s * RPT

    for j in range(RPT // CS):
        pltpu.sync_copy(zeros_hbm, acc.at[pl.ds(base + j * CS, CS)])
    plsc.subcore_barrier()

    pltpu.async_copy(eidx_hbm.at[wid, 0], idx_v.at[0], isem)

    def chunk(j, carry):
        slot = lax.rem(j, 2)
        pslot = lax.rem(j + 1, 2)
        pltpu.make_async_copy(eidx_hbm.at[wid, j], idx_v.at[slot], isem).wait()
        pltpu.async_copy(table_hbm.at[idx_v.at[slot, 0]], rows_v.at[slot], gsem)

        @pl.when(j > 0)
        def _():
            pltpu.make_async_copy(table_hbm.at[idx_v.at[pslot, 0]],
                                  rows_v.at[pslot], gsem).wait()
            pltpu.sync_copy(rows_v.at[pslot], acc.at[idx_v.at[pslot, 1]],
                            add=True)

        @pl.when(j + 1 < NCHS)
        def _():
            pltpu.async_copy(eidx_hbm.at[wid, j + 1], idx_v.at[pslot], isem)

        return carry

    lax.fori_loop(0, NCHS, chunk, 0)
    lslot = (NCHS - 1) % 2
    pltpu.make_async_copy(table_hbm.at[idx_v.at[lslot, 0]],
                          rows_v.at[lslot], gsem).wait()
    pltpu.sync_copy(rows_v.at[lslot], acc.at[idx_v.at[lslot, 1]], add=True)

    plsc.subcore_barrier()
    pltpu.sync_copy(acc.at[pl.ds(base, RPT)],
                    out_hbm.at[pl.ds(c * NPAD + base, RPT)])


def _scat_call(table, eidx, zeros128):
    f = pl.kernel(
        _scat_body,
        out_type=jax.ShapeDtypeStruct((NC * NPAD, D), _f32),
        mesh=_sc_mesh(),
        scratch_types=[
            pltpu.VMEM((2, 2, CS), jnp.int32),
            pltpu.VMEM((2, CS, D), _f32),
            pltpu.SemaphoreType.DMA,
            pltpu.SemaphoreType.DMA,
            pltpu.VMEM_SHARED((NPAD, D), _f32),
        ],
    )
    return f(table, eidx, zeros128)


HB = NPAD // 2
HBP = HB + 16
NV = EPT // 16
NRED = HB // 16


def _deg_body(ei_hbm, out_hbm, sidx_v, didx_v, hist_v, red_v):
    c = lax.axis_index("c")
    s = lax.axis_index("s")
    wid = s * NC + c

    pltpu.sync_copy(ei_hbm.at[pl.ds(wid * EPT, EPT)], sidx_v)
    pltpu.sync_copy(ei_hbm.at[pl.ds(E + wid * EPT, EPT)], didx_v)

    lane = lax.iota(jnp.int32, 16)
    ones16 = jnp.ones((16,), _f32)
    zeros16 = jnp.zeros((16,), _f32)

    for d in range(2):
        idx_ref = sidx_v if d == 0 else didx_v
        for h in range(2):
            lo = h * HB

            def zero(i, carry):
                for u in range(8):
                    hist_v[pl.ds(i * 128 + u * 16, 16)] = zeros16
                return carry

            lax.fori_loop(0, HBP * 16 // 128, zero, 0)

            def cnt(i, carry):
                idx = idx_ref[pl.ds(i * 16, 16)]
                m = (idx >= lo) & (idx < lo + HB)
                pos = lane * HBP + jnp.where(m, idx - lo, HB)
                cur = plsc.load_gather(hist_v, [pos])
                plsc.store_scatter(hist_v, [pos], cur + ones16)
                return carry

            lax.fori_loop(0, NV, cnt, 0)

            def red(i, carry):
                vals = [hist_v[pl.ds(cc * HBP + i * 16, 16)]
                        for cc in range(16)]
                while len(vals) > 1:
                    vals = [vals[k] + vals[k + 1]
                            for k in range(0, len(vals), 2)]
                red_v[pl.ds(i * 16, 16)] = vals[0]
                return carry

            lax.fori_loop(0, NRED, red, 0)
            pltpu.sync_copy(red_v, out_hbm.at[pl.ds((d * NW + wid) * NPAD + lo, HB)])


def _deg_call(edge_index):
    f = pl.kernel(
        _deg_body,
        out_type=jax.ShapeDtypeStruct((2 * NW * NPAD,), _f32),
        mesh=_sc_mesh(),
        compiler_params=pltpu.CompilerParams(needs_layout_passes=False),
        scratch_types=[
            pltpu.VMEM((EPT,), jnp.int32),
            pltpu.VMEM((EPT,), jnp.int32),
            pltpu.VMEM((HBP * 16,), _f32),
            pltpu.VMEM((HB,), _f32),
        ],
    )
    return f(edge_index.reshape(2 * E)).reshape(2, NW, NPAD)


def _prep_body(x_ref, dego_ref, out_ref):
    dego = jnp.sum(dego_ref[...], axis=0) + 1.0
    nsrc = lax.rsqrt(dego)
    out_ref[...] = x_ref[...] * nsrc[:, None]


def _prep_call(x, dego_p):
    return pl.pallas_call(
        _prep_body,
        out_shape=jax.ShapeDtypeStruct((N, D), _f32),
    )(x, dego_p)


def _dense_mid_body(sp_ref, hp_ref, dego_ref, degi_ref, w_ref, b_ref,
                    g_ref, bb_ref, out_ref):
    degi = jnp.sum(degi_ref[...], axis=0) + 1.0
    ndst = lax.rsqrt(degi)
    agg = (sp_ref[0, :N, :] + sp_ref[1, :N, :] + hp_ref[...]) * ndst[:, None]
    z = jnp.dot(agg, w_ref[...], preferred_element_type=_f32) + b_ref[...][None, :]
    r = jnp.maximum(z, 0.0)
    m = jnp.mean(r, axis=0)
    v = jnp.mean(r * r, axis=0) - m * m
    h = (r - m) * lax.rsqrt(v + 1e-5) * g_ref[...][None, :] + bb_ref[...][None, :]
    dego = jnp.sum(dego_ref[...], axis=0) + 1.0
    nsrc = lax.rsqrt(dego)
    out_ref[...] = h * nsrc[:, None]


def _dense_mid_call(sp, hp, dego_p, degi_p, w, b, g, bb):
    sp = sp.reshape(NC, NPAD, D)
    return pl.pallas_call(
        _dense_mid_body,
        out_shape=jax.ShapeDtypeStruct((N, D), _f32),
    )(sp, hp, dego_p, degi_p, w, b, g, bb)


def _dense_fin_body(sp_ref, hp_ref, degi_ref, w_ref, b_ref,
                    g_ref, bb_ref, out_ref):
    degi = jnp.sum(degi_ref[...], axis=0) + 1.0
    ndst = lax.rsqrt(degi)
    agg = (sp_ref[0, :N, :] + sp_ref[1, :N, :] + hp_ref[...]) * ndst[:, None]
    z = jnp.dot(agg, w_ref[...], preferred_element_type=_f32) + b_ref[...][None, :]
    m = jnp.mean(z, axis=-1, keepdims=True)
    zc = z - m
    v = jnp.mean(zc * zc, axis=-1, keepdims=True)
    out_ref[...] = zc * lax.rsqrt(v + 1e-5) * g_ref[...][None, :] + bb_ref[...][None, :]


def _dense_fin_call(sp, hp, degi_p, w, b, g, bb):
    sp = sp.reshape(NC, NPAD, D)
    return pl.pallas_call(
        _dense_fin_body,
        out_shape=jax.ShapeDtypeStruct((N, D), _f32),
    )(sp, hp, degi_p, w, b, g, bb)


def kernel(x, edge_index, W1, b1, W2, b2, W3, b3,
           bn1_g, bn1_b, bn2_g, bn2_b, ln_g, ln_b):
    src_p = jnp.concatenate([edge_index[0],
                             jnp.arange(EPAD, dtype=jnp.int32) * 2 + 1])
    dst_p = jnp.concatenate([edge_index[1],
                             N + jnp.arange(EPAD, dtype=jnp.int32)
                             % (NPAD - N)])
    eidx = jnp.stack([src_p.reshape(NW, NCHS, CS),
                      dst_p.reshape(NW, NCHS, CS)], axis=2)
    zeros128 = jnp.zeros((CS, D), _f32)

    deg_p = _deg_call(edge_index)
    dego_p = deg_p[0, :, :N]
    degi_p = deg_p[1, :, :N]

    h1p = _prep_call(x, dego_p)
    s1 = _scat_call(h1p, eidx, zeros128)
    h2p = _dense_mid_call(s1, h1p, dego_p, degi_p, W1, b1, bn1_g, bn1_b)
    s2 = _scat_call(h2p, eidx, zeros128)
    h3p = _dense_mid_call(s2, h2p, dego_p, degi_p, W2, b2, bn2_g, bn2_b)
    s3 = _scat_call(h3p, eidx, zeros128)
    return _dense_fin_call(s3, h3p, degi_p, W3, b3, ln_g, ln_b)


# speedup vs baseline: 1.9418x; 1.0080x over previous
import functools

import jax
import jax.numpy as jnp
from jax import lax
from jax.experimental import pallas as pl
from jax.experimental.pallas import tpu as pltpu
from jax.experimental.pallas import tpu_sc as plsc

N = 10000
D = 128
E = 320000

NC = 2
NS = 16
NW = NC * NS
EPT = E // NW
C = 80
NCH = EPT // C
CS = 128
NCHS = 79
EPAD = NW * NCHS * CS - E
NPAD = 10240
RPT = NPAD // NS
ZCP = RPT // C

_f32 = jnp.float32


def _sc_mesh():
    return plsc.VectorSubcoreMesh(core_axis_name="c", subcore_axis_name="s",
                                  num_cores=NC, num_subcores=NS)


def _scat_body(table_hbm, eidx_hbm, zeros_hbm,
               out_hbm,
               idx_v, rows_v, isem, gsem, zsem,
               acc):
    c = lax.axis_index("c")
    s = lax.axis_index("s")
    wid = s * NC + c
    base = s * RPT

    for j in range(RPT // CS):
        pltpu.async_copy(zeros_hbm, acc.at[pl.ds(base + j * CS, CS)], zsem)
    pltpu.async_copy(eidx_hbm.at[wid, 0], idx_v.at[0], isem)
    pltpu.make_async_copy(eidx_hbm.at[wid, 0], idx_v.at[0], isem).wait()
    pltpu.async_copy(table_hbm.at[idx_v.at[0, 0]], rows_v.at[0], gsem)
    pltpu.async_copy(eidx_hbm.at[wid, 1], idx_v.at[1], isem)
    for j in range(RPT // CS):
        pltpu.make_async_copy(zeros_hbm, acc.at[pl.ds(base + j * CS, CS)],
                              zsem).wait()
    plsc.subcore_barrier()

    def chunk(j, carry):
        slot = lax.rem(j, 2)
        nslot = lax.rem(j + 1, 2)

        @pl.when(j + 1 < NCHS)
        def _():
            pltpu.make_async_copy(eidx_hbm.at[wid, j + 1], idx_v.at[nslot],
                                  isem).wait()
            pltpu.async_copy(table_hbm.at[idx_v.at[nslot, 0]],
                             rows_v.at[nslot], gsem)

        pltpu.make_async_copy(table_hbm.at[idx_v.at[slot, 0]],
                              rows_v.at[slot], gsem).wait()
        pltpu.sync_copy(rows_v.at[slot], acc.at[idx_v.at[slot, 1]], add=True)

        @pl.when(j + 2 < NCHS)
        def _():
            pltpu.async_copy(eidx_hbm.at[wid, j + 2], idx_v.at[slot], isem)

        return carry

    lax.fori_loop(0, NCHS, chunk, 0)
    plsc.subcore_barrier()
    pltpu.sync_copy(acc.at[pl.ds(base, RPT)],
                    out_hbm.at[pl.ds(c * NPAD + base, RPT)])


def _scat_call(table, eidx, zeros128):
    f = pl.kernel(
        _scat_body,
        out_type=jax.ShapeDtypeStruct((NC * NPAD, D), _f32),
        mesh=_sc_mesh(),
        scratch_types=[
            pltpu.VMEM((2, 2, CS), jnp.int32),
            pltpu.VMEM((2, CS, D), _f32),
            pltpu.SemaphoreType.DMA,
            pltpu.SemaphoreType.DMA,
            pltpu.SemaphoreType.DMA,
            pltpu.VMEM_SHARED((NPAD, D), _f32),
        ],
    )
    return f(table, eidx, zeros128)


HB = NPAD // 2
HBP = HB + 16
NV = EPT // 16
NRED = HB // 16


def _deg_body(ei_hbm, out_hbm, sidx_v, didx_v, hist_v, red_v):
    c = lax.axis_index("c")
    s = lax.axis_index("s")
    wid = s * NC + c

    pltpu.sync_copy(ei_hbm.at[pl.ds(wid * EPT, EPT)], sidx_v)
    pltpu.sync_copy(ei_hbm.at[pl.ds(E + wid * EPT, EPT)], didx_v)

    lane = lax.iota(jnp.int32, 16)
    ones16 = jnp.ones((16,), _f32)
    zeros16 = jnp.zeros((16,), _f32)

    for d in range(2):
        idx_ref = sidx_v if d == 0 else didx_v
        for h in range(2):
            lo = h * HB

            def zero(i, carry):
                for u in range(8):
                    hist_v[pl.ds(i * 128 + u * 16, 16)] = zeros16
                return carry

            lax.fori_loop(0, HBP * 16 // 128, zero, 0)

            def cnt(i, carry):
                idx = idx_ref[pl.ds(i * 16, 16)]
                m = (idx >= lo) & (idx < lo + HB)
                pos = lane * HBP + jnp.where(m, idx - lo, HB)
                cur = plsc.load_gather(hist_v, [pos])
                plsc.store_scatter(hist_v, [pos], cur + ones16)
                return carry

            lax.fori_loop(0, NV, cnt, 0)

            def red(i, carry):
                vals = [hist_v[pl.ds(cc * HBP + i * 16, 16)]
                        for cc in range(16)]
                while len(vals) > 1:
                    vals = [vals[k] + vals[k + 1]
                            for k in range(0, len(vals), 2)]
                red_v[pl.ds(i * 16, 16)] = vals[0]
                return carry

            lax.fori_loop(0, NRED, red, 0)
            pltpu.sync_copy(red_v, out_hbm.at[pl.ds((d * NW + wid) * NPAD + lo, HB)])


def _deg_call(edge_index):
    f = pl.kernel(
        _deg_body,
        out_type=jax.ShapeDtypeStruct((2 * NW * NPAD,), _f32),
        mesh=_sc_mesh(),
        compiler_params=pltpu.CompilerParams(needs_layout_passes=False),
        scratch_types=[
            pltpu.VMEM((EPT,), jnp.int32),
            pltpu.VMEM((EPT,), jnp.int32),
            pltpu.VMEM((HBP * 16,), _f32),
            pltpu.VMEM((HB,), _f32),
        ],
    )
    return f(edge_index.reshape(2 * E)).reshape(2, NW, NPAD)


def _prep_body(x_ref, dego_ref, out_ref):
    dego = jnp.sum(dego_ref[...], axis=0) + 1.0
    nsrc = lax.rsqrt(dego)
    out_ref[...] = x_ref[...] * nsrc[:, None]


def _prep_call(x, dego_p):
    return pl.pallas_call(
        _prep_body,
        out_shape=jax.ShapeDtypeStruct((N, D), _f32),
    )(x, dego_p)


def _dense_mid_body(sp_ref, hp_ref, dego_ref, degi_ref, w_ref, b_ref,
                    g_ref, bb_ref, out_ref):
    degi = jnp.sum(degi_ref[...], axis=0) + 1.0
    ndst = lax.rsqrt(degi)
    agg = (sp_ref[0, :N, :] + sp_ref[1, :N, :] + hp_ref[...]) * ndst[:, None]
    z = jnp.dot(agg, w_ref[...], preferred_element_type=_f32) + b_ref[...][None, :]
    r = jnp.maximum(z, 0.0)
    m = jnp.mean(r, axis=0)
    v = jnp.mean(r * r, axis=0) - m * m
    h = (r - m) * lax.rsqrt(v + 1e-5) * g_ref[...][None, :] + bb_ref[...][None, :]
    dego = jnp.sum(dego_ref[...], axis=0) + 1.0
    nsrc = lax.rsqrt(dego)
    out_ref[...] = h * nsrc[:, None]


def _dense_mid_call(sp, hp, dego_p, degi_p, w, b, g, bb):
    sp = sp.reshape(NC, NPAD, D)
    return pl.pallas_call(
        _dense_mid_body,
        out_shape=jax.ShapeDtypeStruct((N, D), _f32),
    )(sp, hp, dego_p, degi_p, w, b, g, bb)


def _dense_fin_body(sp_ref, hp_ref, degi_ref, w_ref, b_ref,
                    g_ref, bb_ref, out_ref):
    degi = jnp.sum(degi_ref[...], axis=0) + 1.0
    ndst = lax.rsqrt(degi)
    agg = (sp_ref[0, :N, :] + sp_ref[1, :N, :] + hp_ref[...]) * ndst[:, None]
    z = jnp.dot(agg, w_ref[...], preferred_element_type=_f32) + b_ref[...][None, :]
    m = jnp.mean(z, axis=-1, keepdims=True)
    zc = z - m
    v = jnp.mean(zc * zc, axis=-1, keepdims=True)
    out_ref[...] = zc * lax.rsqrt(v + 1e-5) * g_ref[...][None, :] + bb_ref[...][None, :]


def _dense_fin_call(sp, hp, degi_p, w, b, g, bb):
    sp = sp.reshape(NC, NPAD, D)
    return pl.pallas_call(
        _dense_fin_body,
        out_shape=jax.ShapeDtypeStruct((N, D), _f32),
    )(sp, hp, degi_p, w, b, g, bb)


def kernel(x, edge_index, W1, b1, W2, b2, W3, b3,
           bn1_g, bn1_b, bn2_g, bn2_b, ln_g, ln_b):
    src_p = jnp.concatenate([edge_index[0],
                             jnp.arange(EPAD, dtype=jnp.int32) * 2 + 1])
    dst_p = jnp.concatenate([edge_index[1],
                             N + jnp.arange(EPAD, dtype=jnp.int32)
                             % (NPAD - N)])
    eidx = jnp.stack([src_p.reshape(NW, NCHS, CS),
                      dst_p.reshape(NW, NCHS, CS)], axis=2)
    zeros128 = jnp.zeros((CS, D), _f32)

    deg_p = _deg_call(edge_index)
    dego_p = deg_p[0, :, :N]
    degi_p = deg_p[1, :, :N]

    h1p = _prep_call(x, dego_p)
    s1 = _scat_call(h1p, eidx, zeros128)
    h2p = _dense_mid_call(s1, h1p, dego_p, degi_p, W1, b1, bn1_g, bn1_b)
    s2 = _scat_call(h2p, eidx, zeros128)
    h3p = _dense_mid_call(s2, h2p, dego_p, degi_p, W2, b2, bn2_g, bn2_b)
    s3 = _scat_call(h3p, eidx, zeros128)
    return _dense_fin_call(s3, h3p, degi_p, W3, b3, ln_g, ln_b)
